# baseline (speedup 1.0000x reference)
import functools

import jax
import jax.numpy as jnp
from jax import lax
from jax.experimental import pallas as pl
from jax.experimental.pallas import tpu as pltpu
from jax.experimental.pallas import tpu_sc as plsc

N = 10000
D = 128
H = 64

E = 320000
NC = 2
NS = 16
NW = NC * NS
EPW = E // NW
C = 125
CH = EPW // C
RPT = 624
RTL = N - NS * RPT
CL = 200
CP = EPW // CL
NB = 5
GB = 3
SB = NB - GB
NG = CP // NB

CH0 = E // NS // C
NB0 = 5
GB0 = 3
SB0 = NB0 - GB0
NG0 = CH0 // NB0


def _ring_loop(tbl, src_v, dst_v, rows_v, acc, gsem, ssem, ch, nb, gb, ng,
               cl=None):
    sb = nb - gb

    def idx(v, j):
        return v.at[j] if cl is None else v.at[pl.ds(j * cl, cl)]

    for b in range(gb):
        pltpu.async_copy(tbl.at[idx(src_v, b)], rows_v.at[b], gsem[b])
    plsc.subcore_barrier()

    def group(g, carry):
        for b in range(nb):
            j = g * nb + b
            pltpu.make_async_copy(tbl.at[idx(src_v, j)], rows_v.at[b],
                                  gsem[b]).wait()
            pltpu.async_copy(rows_v.at[b], acc.at[idx(dst_v, j)], ssem[b],
                             add=True)
            b2 = (b + gb) % nb
            jo = j - sb
            jn = j + gb
            @pl.when(jo >= 0)
            def _():
                pltpu.make_async_copy(rows_v.at[b2], acc.at[idx(dst_v, jo)],
                                      ssem[b2]).wait()
            @pl.when(jn < ch)
            def _():
                pltpu.async_copy(tbl.at[idx(src_v, jn)], rows_v.at[b2],
                                 gsem[b2])
        return carry

    lax.fori_loop(0, ng, group, 0, unroll=False)
    for k in range(sb):
        j = ch - sb + k
        b = j % nb
        pltpu.make_async_copy(rows_v.at[b], acc.at[idx(dst_v, j)],
                              ssem[b]).wait()
    plsc.subcore_barrier()


def _zero_acc(z_hbm, acc, s):
    pltpu.sync_copy(z_hbm.at[pl.ds(s * RPT, RPT)], acc.at[pl.ds(s * RPT, RPT)])
    @pl.when(s == NS - 1)
    def _():
        pltpu.sync_copy(z_hbm.at[pl.ds(NS * RPT, RTL)],
                        acc.at[pl.ds(NS * RPT, RTL)])


def _copy_out(acc, out_hbm, c, s):
    pltpu.sync_copy(acc.at[pl.ds(s * RPT, RPT)],
                    out_hbm.at[c, pl.ds(s * RPT, RPT)])
    @pl.when(s == NS - 1)
    def _():
        pltpu.sync_copy(acc.at[pl.ds(NS * RPT, RTL)],
                        out_hbm.at[c, pl.ds(NS * RPT, RTL)])


def _agg_sc(h, src2d, dst2d, zeros):
    mesh = plsc.VectorSubcoreMesh(core_axis_name="c", subcore_axis_name="s")

    @functools.partial(
        pl.kernel,
        mesh=mesh,
        out_type=jax.ShapeDtypeStruct((NC, N, H), jnp.float32),
        compiler_params=pltpu.CompilerParams(use_tc_tiling_on_sc=False),
        scratch_types=[
            pltpu.VMEM((EPW,), jnp.int32),
            pltpu.VMEM((EPW,), jnp.int32),
            pltpu.VMEM((NB, CL, H), jnp.float32),
            pltpu.VMEM_SHARED((N, H), jnp.float32),
        ]
        + [pltpu.SemaphoreType.DMA] * (2 * NB),
    )
    def agg(h_hbm, src_hbm, dst_hbm, z_hbm, out_hbm, src_v, dst_v, rows_v, acc,
            *sems):
        gsem = sems[:NB]
        ssem = sems[NB:]
        c = lax.axis_index("c")
        s = lax.axis_index("s")
        w = c * NS + s

        _zero_acc(z_hbm, acc, s)
        pltpu.sync_copy(src_hbm.at[pl.ds(w * EPW, EPW)], src_v)
        pltpu.sync_copy(dst_hbm.at[pl.ds(w * EPW, EPW)], dst_v)
        _ring_loop(h_hbm, src_v, dst_v, rows_v, acc, gsem, ssem, CP, NB, GB, NG,
                   cl=CL)
        _copy_out(acc, out_hbm, c, s)

    return agg(h, src2d, dst2d, zeros)


def _agg0_sc(xcat, src2d, srcp2d, dst2d, zeros):
    mesh = plsc.VectorSubcoreMesh(core_axis_name="c", subcore_axis_name="s")

    @functools.partial(
        pl.kernel,
        mesh=mesh,
        out_type=jax.ShapeDtypeStruct((NC, N, H), jnp.float32),
        compiler_params=pltpu.CompilerParams(use_tc_tiling_on_sc=False),
        scratch_types=[
            pltpu.VMEM((CH0, C), jnp.int32),
            pltpu.VMEM((CH0, C), jnp.int32),
            pltpu.VMEM((NB0, C, H), jnp.float32),
            pltpu.VMEM_SHARED((N, H), jnp.float32),
        ]
        + [pltpu.SemaphoreType.DMA] * (2 * NB0),
    )
    def agg0(x_hbm, src_hbm, srcp_hbm, dst_hbm, z_hbm, out_hbm,
             src_v, dst_v, rows_v, acc, *sems):
        gsem = sems[:NB0]
        ssem = sems[NB0:]
        c = lax.axis_index("c")
        s = lax.axis_index("s")

        _zero_acc(z_hbm, acc, s)
        @pl.when(c == 0)
        def _():
            pltpu.sync_copy(src_hbm.at[pl.ds(s * CH0, CH0)], src_v)
        @pl.when(c == 1)
        def _():
            pltpu.sync_copy(srcp_hbm.at[pl.ds(s * CH0, CH0)], src_v)
        pltpu.sync_copy(dst_hbm.at[pl.ds(s * CH0, CH0)], dst_v)
        _ring_loop(x_hbm, src_v, dst_v, rows_v, acc, gsem, ssem,
                   CH0, NB0, GB0, NG0)
        _copy_out(acc, out_hbm, c, s)

    return agg0(xcat, src2d, srcp2d, dst2d, zeros)


def _aggregate(h, src2d, dst2d, zeros):
    parts = _agg_sc(h, src2d, dst2d, zeros)
    return parts[0], parts[1]


def _layer0_body(h_ref, lr_ref, eps_ref, wa_ref, ba_ref,
                 wb_ref, bb_ref, out_ref):
    agg = jnp.concatenate([lr_ref[0], lr_ref[1]], axis=1)
    z = eps_ref[0, 0] * h_ref[...] + agg
    t = jnp.maximum(jnp.dot(z, wa_ref[...],
                            preferred_element_type=jnp.float32) + ba_ref[...], 0.0)
    u = jnp.dot(t, wb_ref[...], preferred_element_type=jnp.float32) + bb_ref[...]
    out_ref[...] = jnp.maximum(u, 0.0)


_GRID = 10
_BR = N // _GRID


def _bs(shape, mapped_dim0=True):
    if mapped_dim0:
        return pl.BlockSpec(shape, lambda i: (i,) + (0,) * (len(shape) - 1))
    return pl.BlockSpec(shape, lambda i: (0,) * len(shape))


def _layer0(h, lr, eps_i, wa, ba, wb, bb):
    return pl.pallas_call(
        _layer0_body,
        grid=(_GRID,),
        in_specs=[
            _bs((_BR, D)),
            pl.BlockSpec((NC, _BR, H), lambda i: (0, i, 0)),
            _bs((1, 1), False), _bs((D, H), False), _bs((1, H), False),
            _bs((H, H), False), _bs((1, H), False),
        ],
        out_specs=_bs((_BR, H)),
        out_shape=jax.ShapeDtypeStruct((N, H), jnp.float32),
    )(h, lr, eps_i, wa, ba, wb, bb)


def _layer_body(h_ref, a0_ref, a1_ref, eps_ref, wa_ref, ba_ref, wb_ref, bb_ref,
                out_ref):
    z = eps_ref[0, 0] * h_ref[...] + (a0_ref[...] + a1_ref[...])
    t = jnp.maximum(jnp.dot(z, wa_ref[...],
                            preferred_element_type=jnp.float32) + ba_ref[...], 0.0)
    u = jnp.dot(t, wb_ref[...], preferred_element_type=jnp.float32) + bb_ref[...]
    out_ref[...] = jnp.maximum(u, 0.0)


def _layer(h, a0, a1, eps_i, wa, ba, wb, bb):
    return pl.pallas_call(
        _layer_body,
        grid=(_GRID,),
        in_specs=[
            _bs((_BR, H)), _bs((_BR, H)), _bs((_BR, H)),
            _bs((1, 1), False), _bs((H, H), False), _bs((1, H), False),
            _bs((H, H), False), _bs((1, H), False),
        ],
        out_specs=_bs((_BR, H)),
        out_shape=jax.ShapeDtypeStruct((N, H), jnp.float32),
    )(h, a0, a1, eps_i, wa, ba, wb, bb)


def _tail_body(h_ref, a0_ref, a1_ref, eps_ref, wa_ref, ba_ref, wb_ref, bb_ref,
               wha_ref, bha_ref, whb_ref, bhb_ref, out_ref):
    z = eps_ref[0, 0] * h_ref[...] + (a0_ref[...] + a1_ref[...])
    t = jnp.maximum(jnp.dot(z, wa_ref[...],
                            preferred_element_type=jnp.float32) + ba_ref[...], 0.0)
    u = jnp.dot(t, wb_ref[...], preferred_element_type=jnp.float32) + bb_ref[...]
    hn = jnp.maximum(u, 0.0)
    q = jnp.maximum(jnp.dot(hn, wha_ref[...],
                            preferred_element_type=jnp.float32) + bha_ref[...], 0.0)
    out_ref[...] = jnp.dot(q, whb_ref[...],
                           preferred_element_type=jnp.float32) + bhb_ref[...]


def _tail(h, a0, a1, eps_i, wa, ba, wb, bb, wha, bha, whb_p, bhb_p):
    return pl.pallas_call(
        _tail_body,
        grid=(_GRID,),
        in_specs=[
            _bs((_BR, H)), _bs((_BR, H)), _bs((_BR, H)),
            _bs((1, 1), False), _bs((H, H), False), _bs((1, H), False),
            _bs((H, H), False), _bs((1, H), False),
            _bs((H, H // 2), False), _bs((1, H // 2), False),
            _bs((H // 2, 128), False), _bs((1, 128), False),
        ],
        out_specs=_bs((_BR, 128)),
        out_shape=jax.ShapeDtypeStruct((N, 128), jnp.float32),
    )(h, a0, a1, eps_i, wa, ba, wb, bb, wha, bha, whb_p, bhb_p)


def kernel(x, edge_index, eps, w0a, b0a, w0b, b0b, w1a, b1a, w1b, b1b,
           w2a, b2a, w2b, b2b, wha, bha, whb, bhb):
    src1d = edge_index[0]
    dst1d = edge_index[1]
    src2d = edge_index[0].reshape(NW * CH, C)
    dst2d = edge_index[1].reshape(NW * CH, C)
    zH = jnp.zeros((N, H), jnp.float32)

    e0 = (1.0 + eps[0]).reshape(1, 1)
    e1 = (1.0 + eps[1]).reshape(1, 1)
    e2 = (1.0 + eps[2]).reshape(1, 1)

    whb_p = jnp.zeros((H // 2, 128), jnp.float32).at[:, :2].set(whb)
    bhb_p = jnp.zeros((1, 128), jnp.float32).at[:, :2].set(bhb.reshape(1, 2))

    xcat = jnp.concatenate([x[:, :H], x[:, H:]], axis=0)
    srcp2d = src2d + N
    lr = _agg0_sc(xcat, src2d, srcp2d, dst2d, zH)
    h1 = _layer0(x, lr, e0, w0a, b0a.reshape(1, H),
                 w0b, b0b.reshape(1, H))
    a1, a1b = _aggregate(h1, src1d, dst1d, zH)
    h2 = _layer(h1, a1, a1b, e1, w1a, b1a.reshape(1, H), w1b, b1b.reshape(1, H))
    a2, a2b = _aggregate(h2, src1d, dst1d, zH)
    out = _tail(h2, a2, a2b, e2, w2a, b2a.reshape(1, H), w2b, b2b.reshape(1, H),
                wha, bha.reshape(1, H // 2), whb_p, bhb_p)
    return out[:, :2]


# speedup vs baseline: 1.0305x; 1.0305x over previous
import functools

import jax
import jax.numpy as jnp
from jax import lax
from jax.experimental import pallas as pl
from jax.experimental.pallas import tpu as pltpu
from jax.experimental.pallas import tpu_sc as plsc

N = 10000
D = 128
H = 64

E = 320000
NC = 2
NS = 16
NW = NC * NS
EPW = E // NW
C = 125
CH = EPW // C
RPT = 624
RTL = N - NS * RPT
CL = 200
CP = EPW // CL
NB = 5
GB = 3
SB = NB - GB
NG = CP // NB

CH0 = E // NS // C
NB0 = 5
GB0 = 3
SB0 = NB0 - GB0
NG0 = CH0 // NB0


def _ring_loop(tbl, src_v, dst_v, rows_v, acc, gsem, ssem, ch, nb, gb, ng,
               cl=None):
    sb = nb - gb

    def idx(v, j):
        return v.at[j] if cl is None else v.at[pl.ds(j * cl, cl)]

    for b in range(gb):
        pltpu.async_copy(tbl.at[idx(src_v, b)], rows_v.at[b], gsem[b])
    plsc.subcore_barrier()

    def group(g, carry):
        for b in range(nb):
            j = g * nb + b
            pltpu.make_async_copy(tbl.at[idx(src_v, j)], rows_v.at[b],
                                  gsem[b]).wait()
            pltpu.async_copy(rows_v.at[b], acc.at[idx(dst_v, j)], ssem[b],
                             add=True)
            b2 = (b + gb) % nb
            jo = j - sb
            jn = j + gb
            @pl.when(jo >= 0)
            def _():
                pltpu.make_async_copy(rows_v.at[b2], acc.at[idx(dst_v, jo)],
                                      ssem[b2]).wait()
            @pl.when(jn < ch)
            def _():
                pltpu.async_copy(tbl.at[idx(src_v, jn)], rows_v.at[b2],
                                 gsem[b2])
        return carry

    lax.fori_loop(0, ng, group, 0, unroll=False)
    for k in range(sb):
        j = ch - sb + k
        b = j % nb
        pltpu.make_async_copy(rows_v.at[b], acc.at[idx(dst_v, j)],
                              ssem[b]).wait()
    plsc.subcore_barrier()


def _zero_acc(z_hbm, acc, s):
    pltpu.sync_copy(z_hbm.at[pl.ds(s * RPT, RPT)], acc.at[pl.ds(s * RPT, RPT)])
    @pl.when(s == NS - 1)
    def _():
        pltpu.sync_copy(z_hbm.at[pl.ds(NS * RPT, RTL)],
                        acc.at[pl.ds(NS * RPT, RTL)])


def _copy_out(acc, out_hbm, c, s):
    pltpu.sync_copy(acc.at[pl.ds(s * RPT, RPT)],
                    out_hbm.at[c, pl.ds(s * RPT, RPT)])
    @pl.when(s == NS - 1)
    def _():
        pltpu.sync_copy(acc.at[pl.ds(NS * RPT, RTL)],
                        out_hbm.at[c, pl.ds(NS * RPT, RTL)])


def _agg_sc(h, src2d, dst2d, zeros):
    mesh = plsc.VectorSubcoreMesh(core_axis_name="c", subcore_axis_name="s")

    @functools.partial(
        pl.kernel,
        mesh=mesh,
        out_type=jax.ShapeDtypeStruct((NC, N, H), jnp.float32),
        compiler_params=pltpu.CompilerParams(use_tc_tiling_on_sc=False),
        scratch_types=[
            pltpu.VMEM((EPW,), jnp.int32),
            pltpu.VMEM((EPW,), jnp.int32),
            pltpu.VMEM((NB, CL, H), jnp.float32),
            pltpu.VMEM_SHARED((N, H), jnp.float32),
        ]
        + [pltpu.SemaphoreType.DMA] * (2 * NB),
    )
    def agg(h_hbm, src_hbm, dst_hbm, z_hbm, out_hbm, src_v, dst_v, rows_v, acc,
            *sems):
        gsem = sems[:NB]
        ssem = sems[NB:]
        c = lax.axis_index("c")
        s = lax.axis_index("s")
        w = c * NS + s

        _zero_acc(z_hbm, acc, s)
        pltpu.sync_copy(src_hbm.at[pl.ds(w * EPW, EPW)], src_v)
        pltpu.sync_copy(dst_hbm.at[pl.ds(w * EPW, EPW)], dst_v)
        _ring_loop(h_hbm, src_v, dst_v, rows_v, acc, gsem, ssem, CP, NB, GB, NG,
                   cl=CL)
        _copy_out(acc, out_hbm, c, s)

    return agg(h, src2d, dst2d, zeros)


def _agg0_sc(xcat, src2d, srcp2d, dst2d, zeros):
    mesh = plsc.VectorSubcoreMesh(core_axis_name="c", subcore_axis_name="s")

    @functools.partial(
        pl.kernel,
        mesh=mesh,
        out_type=jax.ShapeDtypeStruct((NC, N, H), jnp.float32),
        compiler_params=pltpu.CompilerParams(use_tc_tiling_on_sc=False),
        scratch_types=[
            pltpu.VMEM((CH0, C), jnp.int32),
            pltpu.VMEM((CH0, C), jnp.int32),
            pltpu.VMEM((NB0, C, H), jnp.float32),
            pltpu.VMEM_SHARED((N, H), jnp.float32),
        ]
        + [pltpu.SemaphoreType.DMA] * (2 * NB0),
    )
    def agg0(x_hbm, src_hbm, srcp_hbm, dst_hbm, z_hbm, out_hbm,
             src_v, dst_v, rows_v, acc, *sems):
        gsem = sems[:NB0]
        ssem = sems[NB0:]
        c = lax.axis_index("c")
        s = lax.axis_index("s")

        _zero_acc(z_hbm, acc, s)
        @pl.when(c == 0)
        def _():
            pltpu.sync_copy(src_hbm.at[pl.ds(s * CH0, CH0)], src_v)
        @pl.when(c == 1)
        def _():
            pltpu.sync_copy(srcp_hbm.at[pl.ds(s * CH0, CH0)], src_v)
        pltpu.sync_copy(dst_hbm.at[pl.ds(s * CH0, CH0)], dst_v)
        _ring_loop(x_hbm, src_v, dst_v, rows_v, acc, gsem, ssem,
                   CH0, NB0, GB0, NG0)
        _copy_out(acc, out_hbm, c, s)

    return agg0(xcat, src2d, srcp2d, dst2d, zeros)


def _aggregate(h, src2d, dst2d, zeros):
    parts = _agg_sc(h, src2d, dst2d, zeros)
    return parts[0], parts[1]


def _layer0_body(h_ref, lr_ref, eps_ref, wa_ref, ba_ref,
                 wb_ref, bb_ref, out_ref):
    agg = jnp.concatenate([lr_ref[0], lr_ref[1]], axis=1)
    z = eps_ref[0, 0] * h_ref[...] + agg
    t = jnp.maximum(jnp.dot(z, wa_ref[...],
                            preferred_element_type=jnp.float32) + ba_ref[...], 0.0)
    u = jnp.dot(t, wb_ref[...], preferred_element_type=jnp.float32) + bb_ref[...]
    out_ref[...] = jnp.maximum(u, 0.0)


def _layer0(h, lr, eps_i, wa, ba, wb, bb):
    return pl.pallas_call(
        _layer0_body,
        out_shape=jax.ShapeDtypeStruct((N, H), jnp.float32),
    )(h, lr, eps_i, wa, ba, wb, bb)


def _layer_body(h_ref, a0_ref, a1_ref, eps_ref, wa_ref, ba_ref, wb_ref, bb_ref,
                out_ref):
    z = eps_ref[0, 0] * h_ref[...] + (a0_ref[...] + a1_ref[...])
    t = jnp.maximum(jnp.dot(z, wa_ref[...],
                            preferred_element_type=jnp.float32) + ba_ref[...], 0.0)
    u = jnp.dot(t, wb_ref[...], preferred_element_type=jnp.float32) + bb_ref[...]
    out_ref[...] = jnp.maximum(u, 0.0)


def _layer(h, a0, a1, eps_i, wa, ba, wb, bb):
    return pl.pallas_call(
        _layer_body,
        out_shape=jax.ShapeDtypeStruct((N, H), jnp.float32),
    )(h, a0, a1, eps_i, wa, ba, wb, bb)


def _tail_body(h_ref, a0_ref, a1_ref, eps_ref, wa_ref, ba_ref, wb_ref, bb_ref,
               wha_ref, bha_ref, whb_ref, bhb_ref, out_ref):
    z = eps_ref[0, 0] * h_ref[...] + (a0_ref[...] + a1_ref[...])
    t = jnp.maximum(jnp.dot(z, wa_ref[...],
                            preferred_element_type=jnp.float32) + ba_ref[...], 0.0)
    u = jnp.dot(t, wb_ref[...], preferred_element_type=jnp.float32) + bb_ref[...]
    hn = jnp.maximum(u, 0.0)
    q = jnp.maximum(jnp.dot(hn, wha_ref[...],
                            preferred_element_type=jnp.float32) + bha_ref[...], 0.0)
    out_ref[...] = jnp.dot(q, whb_ref[...],
                           preferred_element_type=jnp.float32) + bhb_ref[...]


def _tail(h, a0, a1, eps_i, wa, ba, wb, bb, wha, bha, whb_p, bhb_p):
    return pl.pallas_call(
        _tail_body,
        out_shape=jax.ShapeDtypeStruct((N, 128), jnp.float32),
    )(h, a0, a1, eps_i, wa, ba, wb, bb, wha, bha, whb_p, bhb_p)


def kernel(x, edge_index, eps, w0a, b0a, w0b, b0b, w1a, b1a, w1b, b1b,
           w2a, b2a, w2b, b2b, wha, bha, whb, bhb):
    src1d = edge_index[0]
    dst1d = edge_index[1]
    src2d = edge_index[0].reshape(NW * CH, C)
    dst2d = edge_index[1].reshape(NW * CH, C)
    zH = jnp.zeros((N, H), jnp.float32)

    e0 = (1.0 + eps[0]).reshape(1, 1)
    e1 = (1.0 + eps[1]).reshape(1, 1)
    e2 = (1.0 + eps[2]).reshape(1, 1)

    whb_p = jnp.zeros((H // 2, 128), jnp.float32).at[:, :2].set(whb)
    bhb_p = jnp.zeros((1, 128), jnp.float32).at[:, :2].set(bhb.reshape(1, 2))

    xcat = jnp.concatenate([x[:, :H], x[:, H:]], axis=0)
    srcp2d = src2d + N
    lr = _agg0_sc(xcat, src2d, srcp2d, dst2d, zH)
    h1 = _layer0(x, lr, e0, w0a, b0a.reshape(1, H),
                 w0b, b0b.reshape(1, H))
    a1, a1b = _aggregate(h1, src1d, dst1d, zH)
    h2 = _layer(h1, a1, a1b, e1, w1a, b1a.reshape(1, H), w1b, b1b.reshape(1, H))
    a2, a2b = _aggregate(h2, src1d, dst1d, zH)
    out = _tail(h2, a2, a2b, e2, w2a, b2a.reshape(1, H), w2b, b2b.reshape(1, H),
                wha, bha.reshape(1, H // 2), whb_p, bhb_p)
    return out[:, :2]
